# baseline (speedup 1.0000x reference)
import functools

import jax
import jax.numpy as jnp
from jax import lax
from jax.experimental import pallas as pl
from jax.experimental.pallas import tpu as pltpu
from jax.experimental.pallas import tpu_sc as plsc

_INFO = plsc.get_sparse_core_info()
_NC = _INFO.num_cores
_NS = _INFO.num_subcores
_NW = _NC * _NS


def _make_gather(num_nodes: int, dim: int, batch: int):
    assert batch % _NW == 0
    bpw = batch // _NW

    mesh = plsc.VectorSubcoreMesh(core_axis_name="c", subcore_axis_name="s")

    @functools.partial(
        pl.kernel,
        mesh=mesh,
        out_type=(
            jax.ShapeDtypeStruct((_NW, bpw, dim), jnp.float32),
            jax.ShapeDtypeStruct((_NW, bpw), jnp.int32),
        ),
        scratch_types=[
            pltpu.VMEM((bpw,), jnp.int32),
            pltpu.VMEM((bpw, dim), jnp.float32),
            pltpu.VMEM((bpw,), jnp.int32),
            pltpu.SemaphoreType.DMA,
            pltpu.SemaphoreType.DMA,
        ],
    )
    def k(mem_hbm, idx_hbm, lu_hbm, mem_out, lu_out,
          idx_v, rows_v, lu_v, sem_m, sem_l):
        wid = lax.axis_index("s") * _NC + lax.axis_index("c")
        pltpu.sync_copy(idx_hbm.at[wid], idx_v)
        m = pltpu.async_copy(mem_hbm.at[idx_v], rows_v, sem_m)
        l = pltpu.async_copy(lu_hbm.at[idx_v], lu_v, sem_l)
        m.wait()
        wm = pltpu.async_copy(rows_v, mem_out.at[wid], sem_m)
        l.wait()
        wl = pltpu.async_copy(lu_v, lu_out.at[wid], sem_l)
        wm.wait()
        wl.wait()

    return k


def kernel(n_id, memory, last_update):
    batch = n_id.shape[0]
    num_nodes, dim = memory.shape
    idx2d = n_id.reshape(_NW, batch // _NW)
    mem3, lu2 = _make_gather(num_nodes, dim, batch)(memory, idx2d, last_update)
    return mem3.reshape(batch, dim), lu2.reshape(batch)
